# baseline (speedup 1.0000x reference)
import functools
import math

import jax
import jax.numpy as jnp
import numpy as np
from jax import lax
from jax.experimental import pallas as pl
from jax.experimental.pallas import tpu as pltpu
from jax.experimental.pallas import tpu_sc as plsc

D_MODEL = 128
CONTEXT = 2048
NC, NS, L = 2, 16, 16
NW = NC * NS
CH = 128


def _make_pe2(context_size, d_model):
    position = np.arange(context_size, dtype=np.float32)[:, None]
    div_term = np.exp(
        np.arange(0, d_model, 2, dtype=np.float32) * (-math.log(10000.0) / d_model)
    )
    pe = np.zeros((context_size, d_model), dtype=np.float32)
    pe[:, 0::2] = np.sin(position * div_term)
    pe[:, 1::2] = np.cos(position * div_term)
    return jnp.asarray(2.0 * pe)


def _embed_body(x_hbm, table_hbm, pe2_hbm, out_hbm, idx_v, rows_v, pe_v, sem):
    b, s_len = x_hbm.shape
    n_chunks = s_len // CH
    w = lax.axis_index("s") * NC + lax.axis_index("c")

    def chunk(c, carry):
        base = c * CH
        pltpu.sync_copy(x_hbm.at[w, pl.ds(base, CH)], idx_v)
        pltpu.sync_copy(pe2_hbm.at[pl.ds(base, CH)], pe_v)
        pltpu.async_copy(table_hbm.at[idx_v], rows_v, sem).wait()

        def row(r, carry2):
            for j in range(D_MODEL // L):
                sl = pl.ds(j * L, L)
                v = rows_v[r, sl]
                rows_v[r, sl] = v + v + pe_v[r, sl]
            return carry2

        lax.fori_loop(0, CH, row, 0)
        pltpu.sync_copy(rows_v, out_hbm.at[w, pl.ds(base, CH)])
        return carry

    lax.fori_loop(0, n_chunks, chunk, 0)


@jax.jit
def kernel(x, table):
    b, s_len = x.shape
    pe2 = _make_pe2(CONTEXT, D_MODEL)[:s_len]
    mesh = plsc.VectorSubcoreMesh(
        core_axis_name="c", subcore_axis_name="s", num_cores=NC, num_subcores=NS
    )
    run = functools.partial(
        pl.kernel,
        out_type=jax.ShapeDtypeStruct((b, s_len, D_MODEL), jnp.float32),
        mesh=mesh,
        scratch_types=[
            pltpu.VMEM((CH,), jnp.int32),
            pltpu.VMEM((CH, D_MODEL), jnp.float32),
            pltpu.VMEM((CH, D_MODEL), jnp.float32),
            pltpu.SemaphoreType.DMA,
        ],
    )(_embed_body)
    return run(x.astype(jnp.int32), table, pe2)


# speedup vs baseline: 1.4192x; 1.4192x over previous
import functools
import math

import jax
import jax.numpy as jnp
import numpy as np
from jax import lax
from jax.experimental import pallas as pl
from jax.experimental.pallas import tpu as pltpu
from jax.experimental.pallas import tpu_sc as plsc

D_MODEL = 128
CONTEXT = 2048
NC, NS, L = 2, 16, 16
NW = NC * NS
NBUF = 4


def _make_pe2(context_size, d_model):
    position = np.arange(context_size, dtype=np.float32)[:, None]
    div_term = np.exp(
        np.arange(0, d_model, 2, dtype=np.float32) * (-math.log(10000.0) / d_model)
    )
    pe = np.zeros((context_size, d_model), dtype=np.float32)
    pe[:, 0::2] = np.sin(position * div_term)
    pe[:, 1::2] = np.cos(position * div_term)
    return jnp.asarray(2.0 * pe)


def _embed_body(xr_hbm, table_hbm, pe2_hbm, out_hbm,
                idx_v, pe_v, rows_v, gsems, wsems):
    _, b_total, sl_len = xr_hbm.shape
    w = lax.axis_index("s") * NC + lax.axis_index("c")
    base = w * sl_len

    pltpu.sync_copy(pe2_hbm.at[pl.ds(base, sl_len)], pe_v)
    pltpu.sync_copy(xr_hbm.at[w], idx_v)

    def gather(b, k):
        pltpu.async_copy(table_hbm.at[idx_v.at[b]], rows_v.at[k], gsems.at[k])

    for k in range(NBUF):
        gather(k, k)

    def outer(g, carry):
        for k in range(NBUF):
            b = g * NBUF + k
            pltpu.make_async_copy(
                table_hbm.at[idx_v.at[b]], rows_v.at[k], gsems.at[k]
            ).wait()

            def row(r, carry2):
                for j in range(D_MODEL // L):
                    sl = pl.ds(j * L, L)
                    rows_v[k, r, sl] = rows_v[k, r, sl] * 2.0 + pe_v[r, sl]
                return carry2

            lax.fori_loop(0, sl_len, row, 0, unroll=2)

            pltpu.async_copy(
                rows_v.at[k], out_hbm.at[b, pl.ds(base, sl_len)], wsems.at[k]
            )
            nb = b + NBUF

            @pl.when(nb < b_total)
            def _():
                pltpu.make_async_copy(
                    rows_v.at[k], out_hbm.at[b, pl.ds(base, sl_len)], wsems.at[k]
                ).wait()
                gather(nb, k)

        return carry

    lax.fori_loop(0, b_total // NBUF, outer, 0)

    for k in range(NBUF):
        b = b_total - NBUF + k
        pltpu.make_async_copy(
            rows_v.at[k], out_hbm.at[b, pl.ds(base, sl_len)], wsems.at[k]
        ).wait()


@jax.jit
def kernel(x, table):
    b, s_len = x.shape
    pe2 = _make_pe2(CONTEXT, D_MODEL)[:s_len]
    sl_len = s_len // NW
    mesh = plsc.VectorSubcoreMesh(
        core_axis_name="c", subcore_axis_name="s", num_cores=NC, num_subcores=NS
    )
    xr = (
        x.astype(jnp.int32)
        .reshape(b, NW, sl_len)
        .transpose(1, 0, 2)
    )
    run = functools.partial(
        pl.kernel,
        out_type=jax.ShapeDtypeStruct((b, s_len, D_MODEL), jnp.float32),
        mesh=mesh,
        scratch_types=[
            pltpu.VMEM((b, sl_len), jnp.int32),
            pltpu.VMEM((sl_len, D_MODEL), jnp.float32),
            pltpu.VMEM((NBUF, sl_len, D_MODEL), jnp.float32),
            pltpu.SemaphoreType.DMA((NBUF,)),
            pltpu.SemaphoreType.DMA((NBUF,)),
        ],
    )(_embed_body)
    return run(xr, table, pe2)


# speedup vs baseline: 1.4269x; 1.0054x over previous
import functools
import math

import jax
import jax.numpy as jnp
import numpy as np
from jax import lax
from jax.experimental import pallas as pl
from jax.experimental.pallas import tpu as pltpu
from jax.experimental.pallas import tpu_sc as plsc

D_MODEL = 128
CONTEXT = 2048
NC, NS, L = 2, 16, 16
NW = NC * NS
NBUF = 4


def _make_pe2(context_size, d_model):
    position = np.arange(context_size, dtype=np.float32)[:, None]
    div_term = np.exp(
        np.arange(0, d_model, 2, dtype=np.float32) * (-math.log(10000.0) / d_model)
    )
    pe = np.zeros((context_size, d_model), dtype=np.float32)
    pe[:, 0::2] = np.sin(position * div_term)
    pe[:, 1::2] = np.cos(position * div_term)
    return jnp.asarray(2.0 * pe)


def _embed_body(xr_hbm, table_hbm, pe2_hbm, out_hbm,
                idx_v, pe_v, gbuf, obuf, gsems, wsems):
    _, b_total, sl_len = xr_hbm.shape
    w = lax.axis_index("s") * NC + lax.axis_index("c")
    base = w * sl_len

    pltpu.sync_copy(pe2_hbm.at[pl.ds(base, sl_len)], pe_v)
    pltpu.sync_copy(xr_hbm.at[w], idx_v)

    def gather(b, k):
        pltpu.async_copy(table_hbm.at[idx_v.at[b]], gbuf.at[k], gsems.at[k])

    for k in range(NBUF):
        gather(k, k)

    def outer(g, carry):
        for k in range(NBUF):
            b = g * NBUF + k
            pltpu.make_async_copy(
                table_hbm.at[idx_v.at[b]], gbuf.at[k], gsems.at[k]
            ).wait()

            @pl.when(b >= NBUF)
            def _():
                pltpu.make_async_copy(
                    obuf.at[k], out_hbm.at[b, pl.ds(base, sl_len)], wsems.at[k]
                ).wait()

            def row(r, carry2):
                for j in range(D_MODEL // L):
                    sl = pl.ds(j * L, L)
                    obuf[k, r, sl] = gbuf[k, r, sl] * 2.0 + pe_v[r, sl]
                return carry2

            lax.fori_loop(0, sl_len, row, 0, unroll=2)

            pltpu.async_copy(
                obuf.at[k], out_hbm.at[b, pl.ds(base, sl_len)], wsems.at[k]
            )
            nb = b + NBUF

            @pl.when(nb < b_total)
            def _():
                gather(nb, k)

        return carry

    lax.fori_loop(0, b_total // NBUF, outer, 0)

    for k in range(NBUF):
        b = b_total - NBUF + k
        pltpu.make_async_copy(
            obuf.at[k], out_hbm.at[b, pl.ds(base, sl_len)], wsems.at[k]
        ).wait()


@jax.jit
def kernel(x, table):
    b, s_len = x.shape
    pe2 = _make_pe2(CONTEXT, D_MODEL)[:s_len]
    sl_len = s_len // NW
    mesh = plsc.VectorSubcoreMesh(
        core_axis_name="c", subcore_axis_name="s", num_cores=NC, num_subcores=NS
    )
    xr = (
        x.astype(jnp.int32)
        .reshape(b, NW, sl_len)
        .transpose(1, 0, 2)
    )
    run = functools.partial(
        pl.kernel,
        out_type=jax.ShapeDtypeStruct((b, s_len, D_MODEL), jnp.float32),
        mesh=mesh,
        scratch_types=[
            pltpu.VMEM((b, sl_len), jnp.int32),
            pltpu.VMEM((sl_len, D_MODEL), jnp.float32),
            pltpu.VMEM((NBUF, sl_len, D_MODEL), jnp.float32),
            pltpu.VMEM((NBUF, sl_len, D_MODEL), jnp.float32),
            pltpu.SemaphoreType.DMA((NBUF,)),
            pltpu.SemaphoreType.DMA((NBUF,)),
        ],
    )(_embed_body)
    return run(xr, table, pe2)


# speedup vs baseline: 1.4636x; 1.0257x over previous
import functools
import math

import jax
import jax.numpy as jnp
import numpy as np
from jax import lax
from jax.experimental import pallas as pl
from jax.experimental.pallas import tpu as pltpu
from jax.experimental.pallas import tpu_sc as plsc

D_MODEL = 128
CONTEXT = 2048
NC, NS, L = 2, 16, 16
NW = NC * NS
NBUF = 4


def _make_pe(context_size, d_model):
    position = np.arange(context_size, dtype=np.float32)[:, None]
    div_term = np.exp(
        np.arange(0, d_model, 2, dtype=np.float32) * (-math.log(10000.0) / d_model)
    )
    pe = np.zeros((context_size, d_model), dtype=np.float32)
    pe[:, 0::2] = np.sin(position * div_term)
    pe[:, 1::2] = np.cos(position * div_term)
    return jnp.asarray(pe)


def _embed_body(xr_hbm, table_hbm, pe_hbm, out_hbm,
                idx_v, pe_sh, gbuf, obuf, gsems, wsems):
    _, b_total, sl_len = xr_hbm.shape
    w = lax.axis_index("s") * NC + lax.axis_index("c")
    sid = lax.axis_index("s")
    base = w * sl_len

    pltpu.sync_copy(pe_hbm.at[pl.ds(base, sl_len)], pe_sh.at[sid])
    pltpu.sync_copy(xr_hbm.at[w], idx_v)

    def gather(b, k):
        pltpu.sync_copy(pe_sh.at[sid], gbuf.at[k])
        pltpu.async_copy(table_hbm.at[idx_v.at[b]], gbuf.at[k], gsems.at[k],
                         add=True)

    for k in range(NBUF):
        gather(k, k)

    def outer(g, carry):
        for k in range(NBUF):
            b = g * NBUF + k
            pltpu.make_async_copy(
                table_hbm.at[idx_v.at[b]], gbuf.at[k], gsems.at[k]
            ).wait()

            @pl.when(b >= NBUF)
            def _():
                pltpu.make_async_copy(
                    obuf.at[k], out_hbm.at[b, pl.ds(base, sl_len)], wsems.at[k]
                ).wait()

            def row(r, carry2):
                for j in range(D_MODEL // L):
                    sl = pl.ds(j * L, L)
                    v = gbuf[k, r, sl]
                    obuf[k, r, sl] = v + v
                return carry2

            lax.fori_loop(0, sl_len, row, 0, unroll=2)

            pltpu.async_copy(
                obuf.at[k], out_hbm.at[b, pl.ds(base, sl_len)], wsems.at[k]
            )
            nb = b + NBUF

            @pl.when(nb < b_total)
            def _():
                gather(nb, k)

        return carry

    lax.fori_loop(0, b_total // NBUF, outer, 0)

    for k in range(NBUF):
        b = b_total - NBUF + k
        pltpu.make_async_copy(
            obuf.at[k], out_hbm.at[b, pl.ds(base, sl_len)], wsems.at[k]
        ).wait()


@jax.jit
def kernel(x, table):
    b, s_len = x.shape
    pe = _make_pe(CONTEXT, D_MODEL)[:s_len]
    sl_len = s_len // NW
    mesh = plsc.VectorSubcoreMesh(
        core_axis_name="c", subcore_axis_name="s", num_cores=NC, num_subcores=NS
    )
    xr = (
        x.astype(jnp.int32)
        .reshape(b, NW, sl_len)
        .transpose(1, 0, 2)
    )
    run = functools.partial(
        pl.kernel,
        out_type=jax.ShapeDtypeStruct((b, s_len, D_MODEL), jnp.float32),
        mesh=mesh,
        scratch_types=[
            pltpu.VMEM((b, sl_len), jnp.int32),
            pltpu.VMEM_SHARED((NS, sl_len, D_MODEL), jnp.float32),
            pltpu.VMEM((NBUF, sl_len, D_MODEL), jnp.float32),
            pltpu.VMEM((NBUF, sl_len, D_MODEL), jnp.float32),
            pltpu.SemaphoreType.DMA((NBUF,)),
            pltpu.SemaphoreType.DMA((NBUF,)),
        ],
    )(_embed_body)
    return run(xr, table, pe)


# speedup vs baseline: 2.3894x; 1.6325x over previous
import functools
import math

import jax
import jax.numpy as jnp
import numpy as np
from jax import lax
from jax.experimental import pallas as pl
from jax.experimental.pallas import tpu as pltpu
from jax.experimental.pallas import tpu_sc as plsc

D_MODEL = 128
CONTEXT = 2048
NC, NS, L = 2, 16, 16
NW = NC * NS
NBUF = 4


def _make_pe2(context_size, d_model):
    position = np.arange(context_size, dtype=np.float32)[:, None]
    div_term = np.exp(
        np.arange(0, d_model, 2, dtype=np.float32) * (-math.log(10000.0) / d_model)
    )
    pe = np.zeros((context_size, d_model), dtype=np.float32)
    pe[:, 0::2] = np.sin(position * div_term)
    pe[:, 1::2] = np.cos(position * div_term)
    return jnp.asarray(2.0 * pe)


def _embed_body(xr_hbm, table_hbm, pe2_hbm, out_hbm,
                idx_v, pe_v, gbuf, obuf, gsems, wsems):
    _, b_total, sl_len = xr_hbm.shape
    w = lax.axis_index("s") * NC + lax.axis_index("c")
    base = w * sl_len

    pltpu.sync_copy(pe2_hbm.at[pl.ds(base, sl_len)], pe_v)
    pltpu.sync_copy(xr_hbm.at[w], idx_v)

    def gather(b, k):
        pltpu.async_copy(table_hbm.at[idx_v.at[b]], gbuf.at[k], gsems.at[k])

    for k in range(NBUF):
        gather(k, k)

    def outer(g, carry):
        for k in range(NBUF):
            b = g * NBUF + k
            pltpu.make_async_copy(
                table_hbm.at[idx_v.at[b]], gbuf.at[k], gsems.at[k]
            ).wait()

            @pl.when(b >= NBUF)
            def _():
                pltpu.make_async_copy(
                    obuf.at[k], out_hbm.at[b, pl.ds(base, sl_len)], wsems.at[k]
                ).wait()

            @plsc.parallel_loop(0, sl_len, unroll=4)
            def _row(r):
                for j in range(D_MODEL // L):
                    sl = pl.ds(j * L, L)
                    obuf[k, r, sl] = gbuf[k, r, sl] * 2.0 + pe_v[r, sl]

            pltpu.async_copy(
                obuf.at[k], out_hbm.at[b, pl.ds(base, sl_len)], wsems.at[k]
            )
            nb = b + NBUF

            @pl.when(nb < b_total)
            def _():
                gather(nb, k)

        return carry

    lax.fori_loop(0, b_total // NBUF, outer, 0)

    for k in range(NBUF):
        b = b_total - NBUF + k
        pltpu.make_async_copy(
            obuf.at[k], out_hbm.at[b, pl.ds(base, sl_len)], wsems.at[k]
        ).wait()


@jax.jit
def kernel(x, table):
    b, s_len = x.shape
    pe2 = _make_pe2(CONTEXT, D_MODEL)[:s_len]
    sl_len = s_len // NW
    mesh = plsc.VectorSubcoreMesh(
        core_axis_name="c", subcore_axis_name="s", num_cores=NC, num_subcores=NS
    )
    xr = (
        x.astype(jnp.int32)
        .reshape(b, NW, sl_len)
        .transpose(1, 0, 2)
    )
    run = functools.partial(
        pl.kernel,
        out_type=jax.ShapeDtypeStruct((b, s_len, D_MODEL), jnp.float32),
        mesh=mesh,
        scratch_types=[
            pltpu.VMEM((b, sl_len), jnp.int32),
            pltpu.VMEM((sl_len, D_MODEL), jnp.float32),
            pltpu.VMEM((NBUF, sl_len, D_MODEL), jnp.float32),
            pltpu.VMEM((NBUF, sl_len, D_MODEL), jnp.float32),
            pltpu.SemaphoreType.DMA((NBUF,)),
            pltpu.SemaphoreType.DMA((NBUF,)),
        ],
    )(_embed_body)
    return run(xr, table, pe2)
